# baseline (speedup 1.0000x reference)
import jax
import jax.numpy as jnp
from jax.experimental import pallas as pl

_NH = 4 * 512
_HB = 128
_GRID = _NH // _HB


def _to_view_colors(colors):
    n, h, w, k, c = colors.shape
    t = colors.transpose(0, 1, 4, 3, 2)
    t = t.reshape(n, h, c, k, w // 128, 128)
    t = t.transpose(0, 1, 2, 4, 3, 5)
    return t.reshape(n * h, c * k, w)


def _to_view_pix(pix):
    n, h, w, k = pix.shape
    t = pix.transpose(0, 1, 3, 2)
    t = t.reshape(n, h, k, w // 128, 128)
    t = t.transpose(0, 1, 3, 2, 4)
    return t.reshape(n * h, k, 512)


def _from_view_out(out, n, h, w):
    t = out.reshape(n, h, 4, 4, 128)
    t = t.transpose(0, 1, 2, 4, 3)
    return t.reshape(n, h, w, 4)


def _shader_body(colors_ref, pix_ref, out_ref):
    cb = colors_ref[...]
    pb = pix_ref[...]
    bg = pb < 0
    c4 = cb.reshape(_HB, 3, 4, 128)
    one = jnp.float32(1.0)
    r = jnp.where(bg, one, c4[:, 0])
    g = jnp.where(bg, one, c4[:, 1])
    b = jnp.where(bg, one, c4[:, 2])
    a = jnp.where(bg, jnp.float32(0.0), one)
    out_ref[...] = jnp.concatenate([r, g, b, a], axis=-1)


def kernel(colors, pix_to_face):
    n, h, w = colors.shape[0], colors.shape[1], colors.shape[2]
    colors_v = _to_view_colors(colors)
    pix_v = _to_view_pix(pix_to_face)
    out = pl.pallas_call(
        _shader_body,
        grid=(_GRID,),
        in_specs=[
            pl.BlockSpec((_HB, 12, 128), lambda i: (i, 0, 0)),
            pl.BlockSpec((_HB, 4, 128), lambda i: (i, 0, 0)),
        ],
        out_specs=pl.BlockSpec((_HB, 4, 512), lambda i: (i, 0, 0)),
        out_shape=jax.ShapeDtypeStruct((_NH, 4, 512), jnp.float32),
    )(colors_v, pix_v)
    return _from_view_out(out, n, h, w)


# speedup vs baseline: 4.5250x; 4.5250x over previous
import functools

import jax
import jax.numpy as jnp
from jax import lax
from jax.experimental import pallas as pl
from jax.experimental.pallas import tpu as pltpu
from jax.experimental.pallas import tpu_sc as plsc

_NH = 4 * 512
_NC, _NS = 2, 16
_NW = _NC * _NS
_RPS = 2
_STEPS = _NH // _NW // _RPS


def _to_view_colors(colors):
    n, h, w, k, c = colors.shape
    t = colors.transpose(0, 1, 4, 3, 2)
    t = t.reshape(n, h, c, k, w // 128, 128)
    t = t.transpose(0, 1, 2, 4, 3, 5)
    return t.reshape(n * h, c, w // 128, k, 128)


def _to_view_pix(pix):
    n, h, w, k = pix.shape
    t = pix.transpose(0, 1, 3, 2)
    t = t.reshape(n, h, k, w // 128, 128)
    t = t.transpose(0, 1, 3, 2, 4)
    return t.reshape(n * h, w // 128, k, 128)


def _from_view_out(out, n, h, w):
    t = out.reshape(n, h, 4, 4, 128)
    t = t.transpose(0, 1, 2, 4, 3)
    return t.reshape(n, h, w, 4)


def _sc_shader(colors_hbm, pix_hbm, out_hbm, stage_c, stage_p, stage_o,
               sem_c, sem_p, sem_o):
    wid = lax.axis_index("s") * _NC + lax.axis_index("c")
    base = wid * (_NH // _NW)

    def step(i, carry):
        nh0 = base + i * _RPS
        cp_c = pltpu.make_async_copy(
            colors_hbm.at[pl.ds(nh0, _RPS), :, :, 0, :], stage_c, sem_c)
        cp_p = pltpu.make_async_copy(
            pix_hbm.at[pl.ds(nh0, _RPS), :, 0, :], stage_p, sem_p)
        cp_c.start()
        cp_p.start()
        cp_c.wait()
        cp_p.wait()
        one = jnp.float32(1.0)
        zero = jnp.float32(0.0)
        for rr in range(_RPS):
            for wt in range(4):
                for g in range(8):
                    sl = pl.ds(g * 16, 16)
                    bg = stage_p[rr, wt, sl] < 0
                    for c in range(3):
                        stage_o[rr, wt, c, sl] = jnp.where(
                            bg, one, stage_c[rr, c, wt, sl])
                    stage_o[rr, wt, 3, sl] = jnp.where(bg, zero, one)
        cp_o = pltpu.make_async_copy(
            stage_o, out_hbm.at[pl.ds(nh0, _RPS)], sem_o)
        cp_o.start()
        cp_o.wait()
        return carry

    lax.fori_loop(0, _STEPS, step, jnp.int32(0))


def kernel(colors, pix_to_face):
    n, h, w = colors.shape[0], colors.shape[1], colors.shape[2]
    colors_v = _to_view_colors(colors)
    pix_v = _to_view_pix(pix_to_face)
    mesh = plsc.VectorSubcoreMesh(core_axis_name="c", subcore_axis_name="s")
    sc_call = functools.partial(
        pl.kernel,
        mesh=mesh,
        out_type=jax.ShapeDtypeStruct((_NH, 4, 4, 128), jnp.float32),
        scratch_types=[
            pltpu.VMEM((_RPS, 3, 4, 128), jnp.float32),
            pltpu.VMEM((_RPS, 4, 128), jnp.int32),
            pltpu.VMEM((_RPS, 4, 4, 128), jnp.float32),
            pltpu.SemaphoreType.DMA,
            pltpu.SemaphoreType.DMA,
            pltpu.SemaphoreType.DMA,
        ],
    )(_sc_shader)
    out = sc_call(colors_v, pix_v)
    return _from_view_out(out, n, h, w)


# speedup vs baseline: 7.3635x; 1.6273x over previous
import functools

import jax
import jax.numpy as jnp
from jax import lax
from jax.experimental import pallas as pl
from jax.experimental.pallas import tpu as pltpu
from jax.experimental.pallas import tpu_sc as plsc

_NH = 4 * 512
_NC, _NS = 2, 16
_NW = _NC * _NS
_RPS = 4
_STEPS = _NH // _NW // _RPS
_HALF = _STEPS // 2


def _to_view_colors(colors):
    n, h, w, k, c = colors.shape
    t = colors.transpose(0, 1, 4, 3, 2)
    t = t.reshape(n, h, c, k, w // 128, 128)
    t = t.transpose(0, 1, 2, 4, 3, 5)
    return t.reshape(n * h, c, w // 128, k, 128)


def _to_view_pix(pix):
    n, h, w, k = pix.shape
    t = pix.transpose(0, 1, 3, 2)
    t = t.reshape(n, h, k, w // 128, 128)
    t = t.transpose(0, 1, 3, 2, 4)
    return t.reshape(n * h, w // 128, k, 128)


def _from_view_out(out, n, h, w):
    t = out.reshape(n, h, 4, 4, 128)
    t = t.transpose(0, 1, 2, 4, 3)
    return t.reshape(n, h, w, 4)


def _sc_shader(colors_hbm, pix_hbm, out_hbm, stage_c, stage_p, stage_o,
               sem_c0, sem_c1, sem_p0, sem_p1, sem_o0, sem_o1):
    wid = lax.axis_index("s") * _NC + lax.axis_index("c")
    base = wid * (_NH // _NW)
    sems = ((sem_c0, sem_p0, sem_o0), (sem_c1, sem_p1, sem_o1))

    def in_copies(step_idx, b):
        nh0 = base + step_idx * _RPS
        sc, sp, _ = sems[b]
        return (
            pltpu.make_async_copy(
                colors_hbm.at[pl.ds(nh0, _RPS), :, :, 0, :], stage_c.at[b], sc),
            pltpu.make_async_copy(
                pix_hbm.at[pl.ds(nh0, _RPS), :, 0, :], stage_p.at[b], sp),
        )

    def out_copy(step_idx, b):
        nh0 = base + step_idx * _RPS
        return pltpu.make_async_copy(
            stage_o.at[b], out_hbm.at[pl.ds(nh0, _RPS)], sems[b][2])

    def compute(b):
        one = jnp.float32(1.0)
        zero = jnp.float32(0.0)
        for rr in range(_RPS):
            for wt in range(4):
                for g in range(8):
                    sl = pl.ds(g * 16, 16)
                    bg = stage_p[b, rr, wt, sl] < 0
                    for c in range(3):
                        stage_o[b, rr, wt, c, sl] = jnp.where(
                            bg, one, stage_c[b, rr, c, wt, sl])
                    stage_o[b, rr, wt, 3, sl] = jnp.where(bg, zero, one)

    def handle(step_idx, b, j):
        cp_c, cp_p = in_copies(step_idx, b)
        cp_c.wait()
        cp_p.wait()
        @pl.when(j > 0)
        def _():
            out_copy(step_idx - 2, b).wait()
        compute(b)
        out_copy(step_idx, b).start()

    for cp in in_copies(0, 0):
        cp.start()

    def body(j, carry):
        i0 = 2 * j
        i1 = 2 * j + 1
        for cp in in_copies(i1, 1):
            cp.start()
        handle(i0, 0, j)
        @pl.when(j + 1 < _HALF)
        def _():
            for cp in in_copies(i0 + 2, 0):
                cp.start()
        handle(i1, 1, j)
        return carry

    lax.fori_loop(0, _HALF, body, jnp.int32(0))
    out_copy(_STEPS - 2, 0).wait()
    out_copy(_STEPS - 1, 1).wait()


def kernel(colors, pix_to_face):
    n, h, w = colors.shape[0], colors.shape[1], colors.shape[2]
    colors_v = _to_view_colors(colors)
    pix_v = _to_view_pix(pix_to_face)
    mesh = plsc.VectorSubcoreMesh(core_axis_name="c", subcore_axis_name="s")
    sc_call = functools.partial(
        pl.kernel,
        mesh=mesh,
        out_type=jax.ShapeDtypeStruct((_NH, 4, 4, 128), jnp.float32),
        scratch_types=[
            pltpu.VMEM((2, _RPS, 3, 4, 128), jnp.float32),
            pltpu.VMEM((2, _RPS, 4, 128), jnp.int32),
            pltpu.VMEM((2, _RPS, 4, 4, 128), jnp.float32),
            pltpu.SemaphoreType.DMA,
            pltpu.SemaphoreType.DMA,
            pltpu.SemaphoreType.DMA,
            pltpu.SemaphoreType.DMA,
            pltpu.SemaphoreType.DMA,
            pltpu.SemaphoreType.DMA,
        ],
    )(_sc_shader)
    out = sc_call(colors_v, pix_v)
    return _from_view_out(out, n, h, w)
